# Initial kernel scaffold; baseline (speedup 1.0000x reference)
import jax
import jax.numpy as jnp
from jax.experimental import pallas as pl


def kernel(meta_xs, node_type, edge_index, edge_type, edge_spatial, Wk, bk, Wq, bq, Wv, bv, Wa, ba, relation_pri, relation_att, relation_msg, skip, ln_g, ln_b):
    raise NotImplementedError("write your pallas kernel here")


# speedup vs baseline: 4.0327x; 4.0327x over previous
import functools

import jax
import jax.numpy as jnp
import numpy as np
from jax import lax
from jax.experimental import pallas as pl
from jax.experimental.pallas import tpu as pltpu
from jax.experimental.pallas import tpu_sc as plsc

N = 10000
E = 160000
D = 128
T = 2
R = 3
H = 8
DK = D // H

NB = 256
N_PAD = 10240
NJ = N_PAD // NB
EB = 64
NW = 32
E_PAD = 163840
EPW = E_PAD // NW
NBLK = EPW // EB
ROWS_PT = N_PAD // 16
DN_ROWS = N_PAD // 8
DROWS_PT = DN_ROWS // 16

_F32 = jnp.float32
_HIGH = jax.lax.Precision.HIGHEST


def _prep_body(x_ref, nt_ref, wq_ref, bq_ref, wk_ref, bk_ref, wv_ref, bv_ref,
               a_ref, m_ref, q_out, kr_out, mr_out):
    x = x_ref[:, :]
    m0 = nt_ref[:, :] == 0.0

    def tproj(w_ref, b_ref):
        p0 = jnp.dot(x, w_ref[0], precision=_HIGH) + b_ref[0:1, :]
        p1 = jnp.dot(x, w_ref[1], precision=_HIGH) + b_ref[1:2, :]
        return jnp.where(m0, p0, p1)

    q_out[:, :] = tproj(wq_ref, bq_ref)
    k = tproj(wk_ref, bk_ref)
    v = tproj(wv_ref, bv_ref)
    kr_out[:, :] = jnp.dot(k, a_ref[0], precision=_HIGH)
    mr_out[:, :] = jnp.dot(v, m_ref[0], precision=_HIGH)


def _edge_body(q_hbm, kr_hbm, mr_hbm, src_hbm, dst_hbm, et_hbm,
               aggr_out, den_out,
               src_v, dst_v, et_v, sidx_v, dstq_v, qrows, krows, mrows,
               wbufw, sem1, sem2, sem3,
               aggr_sh, den_shw):
    c = lax.axis_index("c")
    s = lax.axis_index("s")
    wid = c * 16 + s
    lane = lax.iota(jnp.int32, 16)
    xors = [lane ^ sh for sh in (8, 4, 2, 1)]

    def _allsum(vec):
        for xi in xors:
            vec = vec + vec.at[xi].get(mode="promise_in_bounds")
        return vec

    zeros16 = jnp.zeros((16,), _F32)

    def zrow(i, carry):
        for jj in range(8):
            qrows[i, pl.ds(jj * 16, 16)] = zeros16
            wbufw[i, pl.ds(jj * 16, 16)] = zeros16
        return carry
    lax.fori_loop(0, EB, zrow, 0)

    for chunk in range(ROWS_PT // EB):
        rb = s * ROWS_PT + chunk * EB
        pltpu.sync_copy(qrows, aggr_sh.at[pl.ds(rb, EB), :])
    pltpu.sync_copy(qrows, den_shw.at[pl.ds(s * DROWS_PT, EB), :])
    pltpu.sync_copy(qrows.at[pl.ds(0, DROWS_PT - EB), :],
                    den_shw.at[pl.ds(s * DROWS_PT + EB, DROWS_PT - EB), :])
    plsc.subcore_barrier()

    base_w = wid * EPW

    def blk(j, carry):
        bb = base_w + j * EB
        pltpu.sync_copy(src_hbm.at[pl.ds(bb, EB)], src_v)
        pltpu.sync_copy(dst_hbm.at[pl.ds(bb, EB)], dst_v)
        pltpu.sync_copy(et_hbm.at[pl.ds(bb, EB)], et_v)
        for ii in range(EB // 16):
            sl = pl.ds(ii * 16, 16)
            sidx_v[sl] = et_v[sl] * N_PAD + src_v[sl]
            dstq_v[sl] = lax.shift_right_logical(dst_v[sl], 3)
        for ii in range(EB // 16):
            sl = pl.ds(ii * 16, 16)
            et_v[sl] = dst_v[sl] & 7
        cp1 = pltpu.async_copy(q_hbm.at[dst_v], qrows, sem1)
        cp2 = pltpu.async_copy(kr_hbm.at[sidx_v], krows, sem2)
        cp3 = pltpu.async_copy(mr_hbm.at[sidx_v], mrows, sem3)
        cp1.wait()
        cp2.wait()
        cp3.wait()

        def edge(e, icarry):
            att = zeros16
            for h in range(8):
                sl = pl.ds(h * 16, 16)
                dot = _allsum(qrows[e, sl] * krows[e, sl])
                att = jnp.where(lane == h, dot, att)
            w = jnp.exp(att)
            civ16 = et_v[pl.ds(e & -16, 16)]
            cif = civ16.at[jnp.broadcast_to(e & 15, (16,))].get(
                mode="promise_in_bounds").astype(_F32)
            for m in range(8):
                eqm = jnp.maximum(0.0, 1.0 - jnp.abs(cif - np.float32(m)))
                wbufw[e, pl.ds(m * 16, 16)] = w * eqm
            for h in range(8):
                sl = pl.ds(h * 16, 16)
                mrows[e, sl] = mrows[e, sl] * w[h]
            return icarry
        lax.fori_loop(0, EB, edge, 0)

        pltpu.sync_copy(mrows, aggr_sh.at[dst_v], add=True)
        pltpu.sync_copy(wbufw, den_shw.at[dstq_v], add=True)
        return carry
    lax.fori_loop(0, NBLK, blk, 0)

    plsc.subcore_barrier()
    for chunk in range(ROWS_PT // EB):
        rb = s * ROWS_PT + chunk * EB
        off = c * N_PAD + rb
        pltpu.sync_copy(aggr_sh.at[pl.ds(rb, EB), :], qrows)
        pltpu.sync_copy(qrows, aggr_out.at[pl.ds(off, EB), :])
    drb = s * DROWS_PT
    doff = c * DN_ROWS + drb
    pltpu.sync_copy(den_shw.at[pl.ds(drb, EB), :], qrows)
    pltpu.sync_copy(qrows, den_out.at[pl.ds(doff, EB), :])
    pltpu.sync_copy(den_shw.at[pl.ds(drb + EB, DROWS_PT - EB), :],
                    qrows.at[pl.ds(0, DROWS_PT - EB), :])
    pltpu.sync_copy(qrows.at[pl.ds(0, DROWS_PT - EB), :],
                    den_out.at[pl.ds(doff + EB, DROWS_PT - EB), :])


def _fin_body(aggra_ref, aggrb_ref, dena_ref, denb_ref, x_ref, nt_ref,
              wa_ref, ba_ref, skip_ref, lng_ref, lnb_ref, amat_ref, cmat_ref,
              b8_ref, out_ref):
    m0 = nt_ref[:, :] == 0.0
    dp = dena_ref[:, :] + denb_ref[:, :]
    xfull = jnp.dot(amat_ref[:, :], dp, precision=_HIGH)
    ri = lax.broadcasted_iota(jnp.int32, (NB, 1), 0) & 7
    den8 = jnp.zeros((NB, H), _F32)
    for m in range(8):
        fm = (ri == m).astype(_F32)
        den8 = den8 + jnp.dot(xfull, cmat_ref[m], precision=_HIGH) * fm
    den128 = jnp.dot(den8, b8_ref[:, :], precision=_HIGH) + 1e-16
    aggr = (aggra_ref[:, :] + aggrb_ref[:, :]) / den128
    hg = 0.5 * aggr * (1.0 + lax.erf(aggr * np.float32(1.0 / np.sqrt(2.0))))
    p0 = jnp.dot(hg, wa_ref[0], precision=_HIGH) + ba_ref[0:1, :]
    p1 = jnp.dot(hg, wa_ref[1], precision=_HIGH) + ba_ref[1:2, :]
    trans = jnp.where(m0, p0, p1)
    sg = jax.nn.sigmoid(skip_ref[:, :])
    a = jnp.where(m0, sg[0:1, 0:1], sg[0:1, 1:2])
    x = x_ref[:, :]
    pre = trans * a + x * (1.0 - a)
    mu = jnp.mean(pre, axis=1, keepdims=True)
    var = jnp.mean((pre - mu) ** 2, axis=1, keepdims=True)
    g = jnp.where(m0, lng_ref[0:1, :], lng_ref[1:2, :])
    b = jnp.where(m0, lnb_ref[0:1, :], lnb_ref[1:2, :])
    out_ref[:, :] = (pre - mu) * lax.rsqrt(var + 1e-5) * g + b


def kernel(meta_xs, node_type, edge_index, edge_type, edge_spatial,
           Wk, bk, Wq, bq, Wv, bv, Wa, ba,
           relation_pri, relation_att, relation_msg, skip, ln_g, ln_b):
    del edge_spatial

    x_pad = jnp.pad(meta_xs, ((0, N_PAD - N), (0, 0)))
    nt_f32 = jnp.pad(node_type.astype(_F32), (0, N_PAD - N)).reshape(N_PAD, 1)

    hidx = jnp.arange(H)
    scale = relation_pri / np.float32(np.sqrt(DK))
    A = (jnp.zeros((R, H, DK, H, DK), _F32)
         .at[:, hidx, :, hidx, :]
         .set((relation_att * scale[:, :, None, None]).transpose(1, 0, 2, 3))
         .reshape(R, D, D))
    M = (jnp.zeros((R, H, DK, H, DK), _F32)
         .at[:, hidx, :, hidx, :]
         .set(relation_msg.transpose(1, 0, 2, 3))
         .reshape(R, D, D))

    nblk = pl.BlockSpec((NB, 1), lambda j, r: (j, 0))
    xblk = pl.BlockSpec((NB, D), lambda j, r: (j, 0))
    wblk = pl.BlockSpec((T, D, D), lambda j, r: (0, 0, 0))
    bblk = pl.BlockSpec((T, D), lambda j, r: (0, 0))
    rblk = pl.BlockSpec((1, D, D), lambda j, r: (r, 0, 0))

    q_t, kr_t, mr_t = pl.pallas_call(
        _prep_body,
        grid=(NJ, R),
        in_specs=[xblk, nblk, wblk, bblk, wblk, bblk, wblk, bblk, rblk, rblk],
        out_specs=[
            pl.BlockSpec((NB, D), lambda j, r: (j, 0)),
            pl.BlockSpec((NB, D), lambda j, r: (r * NJ + j, 0)),
            pl.BlockSpec((NB, D), lambda j, r: (r * NJ + j, 0)),
        ],
        out_shape=[
            jax.ShapeDtypeStruct((N_PAD, D), _F32),
            jax.ShapeDtypeStruct((R * N_PAD, D), _F32),
            jax.ShapeDtypeStruct((R * N_PAD, D), _F32),
        ],
    )(x_pad, nt_f32, Wq, bq, Wk, bk, Wv, bv, A, M)

    pad_e = E_PAD - E
    src_p = jnp.concatenate([edge_index[0].astype(jnp.int32),
                             jnp.zeros((pad_e,), jnp.int32)])
    dst_p = jnp.concatenate([edge_index[1].astype(jnp.int32),
                             jnp.full((pad_e,), N_PAD - 1, jnp.int32)])
    et_p = jnp.concatenate([edge_type.astype(jnp.int32),
                            jnp.zeros((pad_e,), jnp.int32)])

    mesh = plsc.VectorSubcoreMesh(core_axis_name="c", subcore_axis_name="s")
    edge_fn = functools.partial(
        pl.kernel,
        out_type=[
            jax.ShapeDtypeStruct((2 * N_PAD, D), _F32),
            jax.ShapeDtypeStruct((2 * DN_ROWS, D), _F32),
        ],
        mesh=mesh,
        scratch_types=[
            pltpu.VMEM((EB,), jnp.int32),
            pltpu.VMEM((EB,), jnp.int32),
            pltpu.VMEM((EB,), jnp.int32),
            pltpu.VMEM((EB,), jnp.int32),
            pltpu.VMEM((EB,), jnp.int32),
            pltpu.VMEM((EB, D), _F32),
            pltpu.VMEM((EB, D), _F32),
            pltpu.VMEM((EB, D), _F32),
            pltpu.VMEM((EB, D), _F32),
            pltpu.SemaphoreType.DMA,
            pltpu.SemaphoreType.DMA,
            pltpu.SemaphoreType.DMA,
            pltpu.VMEM_SHARED((N_PAD, D), _F32),
            pltpu.VMEM_SHARED((DN_ROWS, D), _F32),
        ],
    )(_edge_body)
    aggr2, den2 = edge_fn(q_t, kr_t, mr_t, src_p, dst_p, et_p)

    b8 = jnp.kron(jnp.eye(H, dtype=_F32), jnp.ones((1, DK), _F32))
    amat = (jnp.arange(NB)[:, None] // 8 ==
            jnp.arange(NB // 8)[None, :]).astype(_F32)
    cmat = jnp.zeros((8, D, H), _F32).at[
        jnp.arange(8)[:, None], jnp.arange(8)[:, None] * 16 + jnp.arange(8)[None, :],
        jnp.arange(8)[None, :]].set(1.0)
    skip2 = skip.reshape(1, T)

    out_pad = pl.pallas_call(
        _fin_body,
        grid=(NJ,),
        in_specs=[
            pl.BlockSpec((NB, D), lambda j: (j, 0)),
            pl.BlockSpec((NB, D), lambda j: (NJ + j, 0)),
            pl.BlockSpec((NB // 8, D), lambda j: (j, 0)),
            pl.BlockSpec((NB // 8, D), lambda j: (NJ + j, 0)),
            pl.BlockSpec((NB, D), lambda j: (j, 0)),
            pl.BlockSpec((NB, 1), lambda j: (j, 0)),
            pl.BlockSpec((T, D, D), lambda j: (0, 0, 0)),
            pl.BlockSpec((T, D), lambda j: (0, 0)),
            pl.BlockSpec((1, T), lambda j: (0, 0)),
            pl.BlockSpec((T, D), lambda j: (0, 0)),
            pl.BlockSpec((T, D), lambda j: (0, 0)),
            pl.BlockSpec((NB, NB // 8), lambda j: (0, 0)),
            pl.BlockSpec((8, D, H), lambda j: (0, 0, 0)),
            pl.BlockSpec((H, D), lambda j: (0, 0)),
        ],
        out_specs=pl.BlockSpec((NB, D), lambda j: (j, 0)),
        out_shape=jax.ShapeDtypeStruct((N_PAD, D), _F32),
    )(aggr2, aggr2, den2, den2, x_pad, nt_f32, Wa, ba, skip2, ln_g, ln_b,
      amat, cmat, b8)

    return out_pad[:N, :]


# speedup vs baseline: 4.5108x; 1.1185x over previous
import functools

import jax
import jax.numpy as jnp
import numpy as np
from jax import lax
from jax.experimental import pallas as pl
from jax.experimental.pallas import tpu as pltpu
from jax.experimental.pallas import tpu_sc as plsc

N = 10000
E = 160000
D = 128
T = 2
R = 3
H = 8
DK = D // H

NB = 256
N_PAD = 10240
NJ = N_PAD // NB
EB = 64
NW = 32
E_PAD = 163840
EPW = E_PAD // NW
NBLK = EPW // EB
ROWS_PT = N_PAD // 16
DN_ROWS = N_PAD // 8
DROWS_PT = DN_ROWS // 16

_F32 = jnp.float32
_HIGH = jax.lax.Precision.HIGHEST


def _prep_body(x_ref, nt_ref, wq_ref, bq_ref, wk_ref, bk_ref, wv_ref, bv_ref,
               a_ref, m_ref, q_out, kr_out, mr_out):
    x = x_ref[:, :]
    m0 = nt_ref[:, :] == 0.0

    def tproj(w_ref, b_ref):
        p0 = jnp.dot(x, w_ref[0], precision=_HIGH) + b_ref[0:1, :]
        p1 = jnp.dot(x, w_ref[1], precision=_HIGH) + b_ref[1:2, :]
        return jnp.where(m0, p0, p1)

    q_out[:, :] = tproj(wq_ref, bq_ref)
    k = tproj(wk_ref, bk_ref)
    v = tproj(wv_ref, bv_ref)
    kr_out[:, :] = jnp.dot(k, a_ref[0], precision=_HIGH)
    mr_out[:, :] = jnp.dot(v, m_ref[0], precision=_HIGH)


def _edge_body(q_hbm, kr_hbm, mr_hbm, src_hbm, dst_hbm, et_hbm,
               aggr_out, den_out,
               src_v, dst_v, et_v, sidx_v, dstq_v, qrows, krows, mrows,
               wbufw, sem1, sem2, sem3,
               aggr_sh, den_shw):
    c = lax.axis_index("c")
    s = lax.axis_index("s")
    wid = c * 16 + s
    lane = lax.iota(jnp.int32, 16)
    xors = [lane ^ sh for sh in (8, 4, 2, 1)]

    def _allsum(vec):
        for xi in xors:
            vec = vec + vec.at[xi].get(mode="promise_in_bounds")
        return vec

    zeros16 = jnp.zeros((16,), _F32)

    def zrow(i, carry):
        for jj in range(8):
            qrows[i, pl.ds(jj * 16, 16)] = zeros16
            wbufw[i, pl.ds(jj * 16, 16)] = zeros16
        return carry
    lax.fori_loop(0, EB, zrow, 0)

    for chunk in range(ROWS_PT // EB):
        rb = s * ROWS_PT + chunk * EB
        pltpu.sync_copy(qrows, aggr_sh.at[pl.ds(rb, EB), :])
    pltpu.sync_copy(qrows, den_shw.at[pl.ds(s * DROWS_PT, EB), :])
    pltpu.sync_copy(qrows.at[pl.ds(0, DROWS_PT - EB), :],
                    den_shw.at[pl.ds(s * DROWS_PT + EB, DROWS_PT - EB), :])
    plsc.subcore_barrier()

    base_w = wid * EPW

    def blk(j, carry):
        bb = base_w + j * EB
        pltpu.sync_copy(src_hbm.at[pl.ds(bb, EB)], src_v)
        pltpu.sync_copy(dst_hbm.at[pl.ds(bb, EB)], dst_v)
        pltpu.sync_copy(et_hbm.at[pl.ds(bb, EB)], et_v)
        for ii in range(EB // 16):
            sl = pl.ds(ii * 16, 16)
            sidx_v[sl] = et_v[sl] * N_PAD + src_v[sl]
            dstq_v[sl] = lax.shift_right_logical(dst_v[sl], 3)
        for ii in range(EB // 16):
            sl = pl.ds(ii * 16, 16)
            et_v[sl] = dst_v[sl] & 7
        cp1 = pltpu.async_copy(q_hbm.at[dst_v], qrows, sem1)
        cp2 = pltpu.async_copy(kr_hbm.at[sidx_v], krows, sem2)
        cp3 = pltpu.async_copy(mr_hbm.at[sidx_v], mrows, sem3)
        cp1.wait()
        cp2.wait()
        cp3.wait()

        @plsc.parallel_loop(0, EB, unroll=4)
        def edge(e):
            att = zeros16
            for h in range(8):
                sl = pl.ds(h * 16, 16)
                dot = _allsum(qrows[e, sl] * krows[e, sl])
                att = jnp.where(lane == h, dot, att)
            w = jnp.exp(att)
            civ16 = et_v[pl.ds(e & -16, 16)]
            cif = civ16.at[jnp.broadcast_to(e & 15, (16,))].get(
                mode="promise_in_bounds").astype(_F32)
            for m in range(8):
                eqm = jnp.where(cif == np.float32(m), 1.0, 0.0)
                wbufw[e, pl.ds(m * 16, 16)] = w * eqm
            for h in range(8):
                sl = pl.ds(h * 16, 16)
                mrows[e, sl] = mrows[e, sl] * w[h]

        pltpu.sync_copy(mrows, aggr_sh.at[dst_v], add=True)
        pltpu.sync_copy(wbufw, den_shw.at[dstq_v], add=True)
        return carry
    lax.fori_loop(0, NBLK, blk, 0)

    plsc.subcore_barrier()
    for chunk in range(ROWS_PT // EB):
        rb = s * ROWS_PT + chunk * EB
        off = c * N_PAD + rb
        pltpu.sync_copy(aggr_sh.at[pl.ds(rb, EB), :], qrows)
        pltpu.sync_copy(qrows, aggr_out.at[pl.ds(off, EB), :])
    drb = s * DROWS_PT
    doff = c * DN_ROWS + drb
    pltpu.sync_copy(den_shw.at[pl.ds(drb, EB), :], qrows)
    pltpu.sync_copy(qrows, den_out.at[pl.ds(doff, EB), :])
    pltpu.sync_copy(den_shw.at[pl.ds(drb + EB, DROWS_PT - EB), :],
                    qrows.at[pl.ds(0, DROWS_PT - EB), :])
    pltpu.sync_copy(qrows.at[pl.ds(0, DROWS_PT - EB), :],
                    den_out.at[pl.ds(doff + EB, DROWS_PT - EB), :])


def _fin_body(aggra_ref, aggrb_ref, dena_ref, denb_ref, x_ref, nt_ref,
              wa_ref, ba_ref, skip_ref, lng_ref, lnb_ref, amat_ref, cmat_ref,
              b8_ref, out_ref):
    m0 = nt_ref[:, :] == 0.0
    dp = dena_ref[:, :] + denb_ref[:, :]
    xfull = jnp.dot(amat_ref[:, :], dp, precision=_HIGH)
    ri = lax.broadcasted_iota(jnp.int32, (NB, 1), 0) & 7
    den8 = jnp.zeros((NB, H), _F32)
    for m in range(8):
        fm = (ri == m).astype(_F32)
        den8 = den8 + jnp.dot(xfull, cmat_ref[m], precision=_HIGH) * fm
    den128 = jnp.dot(den8, b8_ref[:, :], precision=_HIGH) + 1e-16
    aggr = (aggra_ref[:, :] + aggrb_ref[:, :]) / den128
    hg = 0.5 * aggr * (1.0 + lax.erf(aggr * np.float32(1.0 / np.sqrt(2.0))))
    p0 = jnp.dot(hg, wa_ref[0], precision=_HIGH) + ba_ref[0:1, :]
    p1 = jnp.dot(hg, wa_ref[1], precision=_HIGH) + ba_ref[1:2, :]
    trans = jnp.where(m0, p0, p1)
    sg = jax.nn.sigmoid(skip_ref[:, :])
    a = jnp.where(m0, sg[0:1, 0:1], sg[0:1, 1:2])
    x = x_ref[:, :]
    pre = trans * a + x * (1.0 - a)
    mu = jnp.mean(pre, axis=1, keepdims=True)
    var = jnp.mean((pre - mu) ** 2, axis=1, keepdims=True)
    g = jnp.where(m0, lng_ref[0:1, :], lng_ref[1:2, :])
    b = jnp.where(m0, lnb_ref[0:1, :], lnb_ref[1:2, :])
    out_ref[:, :] = (pre - mu) * lax.rsqrt(var + 1e-5) * g + b


def kernel(meta_xs, node_type, edge_index, edge_type, edge_spatial,
           Wk, bk, Wq, bq, Wv, bv, Wa, ba,
           relation_pri, relation_att, relation_msg, skip, ln_g, ln_b):
    del edge_spatial

    x_pad = jnp.pad(meta_xs, ((0, N_PAD - N), (0, 0)))
    nt_f32 = jnp.pad(node_type.astype(_F32), (0, N_PAD - N)).reshape(N_PAD, 1)

    hidx = jnp.arange(H)
    scale = relation_pri / np.float32(np.sqrt(DK))
    A = (jnp.zeros((R, H, DK, H, DK), _F32)
         .at[:, hidx, :, hidx, :]
         .set((relation_att * scale[:, :, None, None]).transpose(1, 0, 2, 3))
         .reshape(R, D, D))
    M = (jnp.zeros((R, H, DK, H, DK), _F32)
         .at[:, hidx, :, hidx, :]
         .set(relation_msg.transpose(1, 0, 2, 3))
         .reshape(R, D, D))

    nblk = pl.BlockSpec((NB, 1), lambda j, r: (j, 0))
    xblk = pl.BlockSpec((NB, D), lambda j, r: (j, 0))
    wblk = pl.BlockSpec((T, D, D), lambda j, r: (0, 0, 0))
    bblk = pl.BlockSpec((T, D), lambda j, r: (0, 0))
    rblk = pl.BlockSpec((1, D, D), lambda j, r: (r, 0, 0))

    q_t, kr_t, mr_t = pl.pallas_call(
        _prep_body,
        grid=(NJ, R),
        in_specs=[xblk, nblk, wblk, bblk, wblk, bblk, wblk, bblk, rblk, rblk],
        out_specs=[
            pl.BlockSpec((NB, D), lambda j, r: (j, 0)),
            pl.BlockSpec((NB, D), lambda j, r: (r * NJ + j, 0)),
            pl.BlockSpec((NB, D), lambda j, r: (r * NJ + j, 0)),
        ],
        out_shape=[
            jax.ShapeDtypeStruct((N_PAD, D), _F32),
            jax.ShapeDtypeStruct((R * N_PAD, D), _F32),
            jax.ShapeDtypeStruct((R * N_PAD, D), _F32),
        ],
    )(x_pad, nt_f32, Wq, bq, Wk, bk, Wv, bv, A, M)

    pad_e = E_PAD - E
    src_p = jnp.concatenate([edge_index[0].astype(jnp.int32),
                             jnp.zeros((pad_e,), jnp.int32)])
    dst_p = jnp.concatenate([edge_index[1].astype(jnp.int32),
                             jnp.full((pad_e,), N_PAD - 1, jnp.int32)])
    et_p = jnp.concatenate([edge_type.astype(jnp.int32),
                            jnp.zeros((pad_e,), jnp.int32)])

    mesh = plsc.VectorSubcoreMesh(core_axis_name="c", subcore_axis_name="s")
    edge_fn = functools.partial(
        pl.kernel,
        out_type=[
            jax.ShapeDtypeStruct((2 * N_PAD, D), _F32),
            jax.ShapeDtypeStruct((2 * DN_ROWS, D), _F32),
        ],
        mesh=mesh,
        scratch_types=[
            pltpu.VMEM((EB,), jnp.int32),
            pltpu.VMEM((EB,), jnp.int32),
            pltpu.VMEM((EB,), jnp.int32),
            pltpu.VMEM((EB,), jnp.int32),
            pltpu.VMEM((EB,), jnp.int32),
            pltpu.VMEM((EB, D), _F32),
            pltpu.VMEM((EB, D), _F32),
            pltpu.VMEM((EB, D), _F32),
            pltpu.VMEM((EB, D), _F32),
            pltpu.SemaphoreType.DMA,
            pltpu.SemaphoreType.DMA,
            pltpu.SemaphoreType.DMA,
            pltpu.VMEM_SHARED((N_PAD, D), _F32),
            pltpu.VMEM_SHARED((DN_ROWS, D), _F32),
        ],
    )(_edge_body)
    aggr2, den2 = edge_fn(q_t, kr_t, mr_t, src_p, dst_p, et_p)

    b8 = jnp.kron(jnp.eye(H, dtype=_F32), jnp.ones((1, DK), _F32))
    amat = (jnp.arange(NB)[:, None] // 8 ==
            jnp.arange(NB // 8)[None, :]).astype(_F32)
    cmat = jnp.zeros((8, D, H), _F32).at[
        jnp.arange(8)[:, None], jnp.arange(8)[:, None] * 16 + jnp.arange(8)[None, :],
        jnp.arange(8)[None, :]].set(1.0)
    skip2 = skip.reshape(1, T)

    out_pad = pl.pallas_call(
        _fin_body,
        grid=(NJ,),
        in_specs=[
            pl.BlockSpec((NB, D), lambda j: (j, 0)),
            pl.BlockSpec((NB, D), lambda j: (NJ + j, 0)),
            pl.BlockSpec((NB // 8, D), lambda j: (j, 0)),
            pl.BlockSpec((NB // 8, D), lambda j: (NJ + j, 0)),
            pl.BlockSpec((NB, D), lambda j: (j, 0)),
            pl.BlockSpec((NB, 1), lambda j: (j, 0)),
            pl.BlockSpec((T, D, D), lambda j: (0, 0, 0)),
            pl.BlockSpec((T, D), lambda j: (0, 0)),
            pl.BlockSpec((1, T), lambda j: (0, 0)),
            pl.BlockSpec((T, D), lambda j: (0, 0)),
            pl.BlockSpec((T, D), lambda j: (0, 0)),
            pl.BlockSpec((NB, NB // 8), lambda j: (0, 0)),
            pl.BlockSpec((8, D, H), lambda j: (0, 0, 0)),
            pl.BlockSpec((H, D), lambda j: (0, 0)),
        ],
        out_specs=pl.BlockSpec((NB, D), lambda j: (j, 0)),
        out_shape=jax.ShapeDtypeStruct((N_PAD, D), _F32),
    )(aggr2, aggr2, den2, den2, x_pad, nt_f32, Wa, ba, skip2, ln_g, ln_b,
      amat, cmat, b8)

    return out_pad[:N, :]
